# baseline (speedup 1.0000x reference)
import functools

import jax
import jax.numpy as jnp
import numpy as np
from jax.experimental import pallas as pl
from jax.experimental.pallas import tpu as pltpu
from jax.sharding import Mesh, PartitionSpec as P


def _dot_t(a, w):
    return jax.lax.dot_general(a, w, (((1,), (1,)), ((), ())),
                               preferred_element_type=jnp.float32)


def _lstm_body(H, x_ref, h_ref, c_ref, wx_ref, wh_ref, b_ref, wo_ref,
               bo_ref, out_ref, hnew_ref):
    xb = x_ref[...].astype(jnp.bfloat16)
    hb = h_ref[...].astype(jnp.bfloat16)
    g = _dot_t(xb, wx_ref[...]) + _dot_t(hb, wh_ref[...]) + b_ref[...]
    i = jax.nn.sigmoid(g[:, :H])
    o = jax.nn.sigmoid(g[:, H:2 * H])
    f = jax.nn.sigmoid(g[:, 2 * H:3 * H])
    z = jnp.tanh(g[:, 3 * H:])
    c_new = i * z + f * c_ref[...]
    hn = o * jnp.tanh(c_new)
    hnew_ref[...] = hn
    out_ref[...] = _dot_t(hn.astype(jnp.bfloat16), wo_ref[...]) + bo_ref[...]


def _lstm_block(inp, h, c, WxR, WhR, b, WoB, bo):
    B, I = inp.shape
    H = h.shape[1]
    O = WoB.shape[0]

    BM = 512
    grid = (B // BM,)

    return pl.pallas_call(
        functools.partial(_lstm_body, H),
        grid=grid,
        in_specs=[
            pl.BlockSpec((BM, I), lambda b_: (b_, 0)),
            pl.BlockSpec((BM, H), lambda b_: (b_, 0)),
            pl.BlockSpec((BM, H), lambda b_: (b_, 0)),
            pl.BlockSpec((4 * H, I), lambda b_: (0, 0)),
            pl.BlockSpec((4 * H, H), lambda b_: (0, 0)),
            pl.BlockSpec((1, 4 * H), lambda b_: (0, 0)),
            pl.BlockSpec((O, H), lambda b_: (0, 0)),
            pl.BlockSpec((1, O), lambda b_: (0, 0)),
        ],
        out_specs=[
            pl.BlockSpec((BM, O), lambda b_: (b_, 0)),
            pl.BlockSpec((BM, H), lambda b_: (b_, 0)),
        ],
        out_shape=[
            jax.ShapeDtypeStruct((B, O), jnp.float32),
            jax.ShapeDtypeStruct((B, H), jnp.float32),
        ],
        compiler_params=pltpu.CompilerParams(
            dimension_semantics=("arbitrary",),
            vmem_limit_bytes=64 * 1024 * 1024,
        ),
    )(inp, h, c, WxR, WhR, b, WoB, bo)


@jax.jit
def kernel(inp, h, c, Wx, bx, Wh, Wout, bout):
    H = h.shape[1]
    I = inp.shape[1]
    O = Wout.shape[0]

    WxR = Wx.reshape(4 * H, I).astype(jnp.bfloat16)
    WhR = Wh.reshape(4 * H, H).astype(jnp.bfloat16)
    b = bx.reshape(1, 4 * H)
    WoB = Wout.astype(jnp.bfloat16)
    bo = bout.reshape(1, O)

    devs = jax.devices()
    ndev = len(devs) if inp.shape[0] % (512 * len(devs)) == 0 else 1
    if ndev > 1:
        mesh = Mesh(np.array(devs), ("x",))
        fn = jax.shard_map(
            _lstm_block,
            mesh=mesh,
            check_vma=False,
            in_specs=(P("x", None), P("x", None), P("x", None),
                      P(None, None), P(None, None), P(None, None),
                      P(None, None), P(None, None)),
            out_specs=(P("x", None), P("x", None)),
        )
        out, h_new = fn(inp, h, c, WxR, WhR, b, WoB, bo)
        out = jax.device_put(out, devs[0])
        h_new = jax.device_put(h_new, devs[0])
    else:
        out, h_new = _lstm_block(inp, h, c, WxR, WhR, b, WoB, bo)
    return (out, h_new)


# speedup vs baseline: 2.3409x; 2.3409x over previous
import functools

import jax
import jax.numpy as jnp
from jax.experimental import pallas as pl
from jax.experimental.pallas import tpu as pltpu


def _dot_t(a, w):
    return jax.lax.dot_general(a, w, (((1,), (1,)), ((), ())),
                               preferred_element_type=jnp.float32)


def _lstm_chain(H, rows, x_ref, h_ref, c_ref, wx_ref, wh_ref, b_ref,
                wo_ref, bo_ref, out_ref, hnew_ref):
    xb = x_ref[rows, :].astype(jnp.bfloat16)
    hb = h_ref[rows, :].astype(jnp.bfloat16)
    g = _dot_t(xb, wx_ref[...]) + _dot_t(hb, wh_ref[...]) + b_ref[...]
    i = jax.nn.sigmoid(g[:, :H])
    o = jax.nn.sigmoid(g[:, H:2 * H])
    f = jax.nn.sigmoid(g[:, 2 * H:3 * H])
    z = jnp.tanh(g[:, 3 * H:])
    c_new = i * z + f * c_ref[rows, :]
    hn = o * jnp.tanh(c_new)
    hnew_ref[rows, :] = hn
    out_ref[rows, :] = (_dot_t(hn.astype(jnp.bfloat16), wo_ref[...])
                        + bo_ref[...])


def _lstm_body(H, n_chains, x_ref, h_ref, c_ref, wx_ref, wh_ref, b_ref,
               wo_ref, bo_ref, out_ref, hnew_ref):
    bm = x_ref.shape[0]
    step = bm // n_chains
    for k in range(n_chains):
        rows = slice(k * step, (k + 1) * step)
        _lstm_chain(H, rows, x_ref, h_ref, c_ref, wx_ref, wh_ref, b_ref,
                    wo_ref, bo_ref, out_ref, hnew_ref)


@jax.jit
def kernel(inp, h, c, Wx, bx, Wh, Wout, bout):
    B, I = inp.shape
    H = h.shape[1]
    O = Wout.shape[0]

    WxR = Wx.reshape(4 * H, I).astype(jnp.bfloat16)
    WhR = Wh.reshape(4 * H, H).astype(jnp.bfloat16)
    b = bx.reshape(1, 4 * H)
    WoB = Wout.astype(jnp.bfloat16)
    bo = bout.reshape(1, O)

    BM = 512
    grid = (B // BM,)

    out, h_new = pl.pallas_call(
        functools.partial(_lstm_body, H, 2),
        grid=grid,
        in_specs=[
            pl.BlockSpec((BM, I), lambda b_: (b_, 0)),
            pl.BlockSpec((BM, H), lambda b_: (b_, 0)),
            pl.BlockSpec((BM, H), lambda b_: (b_, 0)),
            pl.BlockSpec((4 * H, I), lambda b_: (0, 0)),
            pl.BlockSpec((4 * H, H), lambda b_: (0, 0)),
            pl.BlockSpec((1, 4 * H), lambda b_: (0, 0)),
            pl.BlockSpec((O, H), lambda b_: (0, 0)),
            pl.BlockSpec((1, O), lambda b_: (0, 0)),
        ],
        out_specs=[
            pl.BlockSpec((BM, O), lambda b_: (b_, 0)),
            pl.BlockSpec((BM, H), lambda b_: (b_, 0)),
        ],
        out_shape=[
            jax.ShapeDtypeStruct((B, O), jnp.float32),
            jax.ShapeDtypeStruct((B, H), jnp.float32),
        ],
        compiler_params=pltpu.CompilerParams(
            dimension_semantics=("parallel",),
            vmem_limit_bytes=64 * 1024 * 1024,
        ),
    )(inp, h, c, WxR, WhR, b, WoB, bo)
    return (out, h_new)


# speedup vs baseline: 2.3863x; 1.0194x over previous
import functools

import jax
import jax.numpy as jnp
from jax.experimental import pallas as pl
from jax.experimental.pallas import tpu as pltpu


def _dot_t(a, w):
    return jax.lax.dot_general(a, w, (((1,), (1,)), ((), ())),
                               preferred_element_type=jnp.float32)


def _lstm_chain(H, rows, x_ref, h_ref, c_ref, wx_ref, wh_ref, b_ref,
                wo_ref, bo_ref, out_ref, hnew_ref):
    xb = x_ref[rows, :].astype(jnp.bfloat16)
    hb = h_ref[rows, :].astype(jnp.bfloat16)
    g = _dot_t(xb, wx_ref[...]) + _dot_t(hb, wh_ref[...]) + b_ref[...]
    i = jax.nn.sigmoid(g[:, :H])
    o = jax.nn.sigmoid(g[:, H:2 * H])
    f = jax.nn.sigmoid(g[:, 2 * H:3 * H])
    z = jnp.tanh(g[:, 3 * H:])
    c_new = i * z + f * c_ref[rows, :]
    hn = o * jnp.tanh(c_new)
    hnew_ref[rows, :] = hn
    out_ref[rows, :] = (_dot_t(hn.astype(jnp.bfloat16), wo_ref[...])
                        + bo_ref[...])


def _lstm_body(H, n_chains, x_ref, h_ref, c_ref, wx_ref, wh_ref, b_ref,
               wo_ref, bo_ref, out_ref, hnew_ref):
    bm = x_ref.shape[0]
    step = bm // n_chains
    for k in range(n_chains):
        rows = slice(k * step, (k + 1) * step)
        _lstm_chain(H, rows, x_ref, h_ref, c_ref, wx_ref, wh_ref, b_ref,
                    wo_ref, bo_ref, out_ref, hnew_ref)


@jax.jit
def kernel(inp, h, c, Wx, bx, Wh, Wout, bout):
    B, I = inp.shape
    H = h.shape[1]
    O = Wout.shape[0]

    WxR = Wx.reshape(4 * H, I).astype(jnp.bfloat16)
    WhR = Wh.reshape(4 * H, H).astype(jnp.bfloat16)
    b = bx.reshape(1, 4 * H)
    WoB = Wout.astype(jnp.bfloat16)
    bo = bout.reshape(1, O)

    BM = 1024
    grid = (B // BM,)

    out, h_new = pl.pallas_call(
        functools.partial(_lstm_body, H, 4),
        grid=grid,
        in_specs=[
            pl.BlockSpec((BM, I), lambda b_: (b_, 0)),
            pl.BlockSpec((BM, H), lambda b_: (b_, 0)),
            pl.BlockSpec((BM, H), lambda b_: (b_, 0)),
            pl.BlockSpec((4 * H, I), lambda b_: (0, 0)),
            pl.BlockSpec((4 * H, H), lambda b_: (0, 0)),
            pl.BlockSpec((1, 4 * H), lambda b_: (0, 0)),
            pl.BlockSpec((O, H), lambda b_: (0, 0)),
            pl.BlockSpec((1, O), lambda b_: (0, 0)),
        ],
        out_specs=[
            pl.BlockSpec((BM, O), lambda b_: (b_, 0)),
            pl.BlockSpec((BM, H), lambda b_: (b_, 0)),
        ],
        out_shape=[
            jax.ShapeDtypeStruct((B, O), jnp.float32),
            jax.ShapeDtypeStruct((B, H), jnp.float32),
        ],
        compiler_params=pltpu.CompilerParams(
            dimension_semantics=("parallel",),
            vmem_limit_bytes=64 * 1024 * 1024,
        ),
    )(inp, h, c, WxR, WhR, b, WoB, bo)
    return (out, h_new)
